# baseline (speedup 1.0000x reference)
import functools

import jax
import jax.numpy as jnp
from jax import lax
from jax.experimental import pallas as pl
from jax.experimental.pallas import tpu as pltpu
from jax.experimental.pallas import tpu_sc as plsc

S, T = 4096, 200
D = 64
SCALE = 8.0
NC, NS, L = 2, 16, 16
NW = NC * NS
SB = S // NW
NG = SB // L


def _emb_body(x_hbm, tab_hbm, out_hbm, idx_v, pidx_v, pbuf, obuf,
              gsems, wsems):
    wid = lax.axis_index("s") * NC + lax.axis_index("c")
    base = wid * SB
    iota = lax.broadcasted_iota(jnp.int32, (L,), 0)
    rows = [iota + g * L for g in range(NG)]

    pltpu.sync_copy(x_hbm.at[pl.ds(base, SB)], idx_v)

    def start_gather(t, s):
        t_vec = jnp.full((L,), 0, jnp.int32) + t
        for g in range(NG):
            pidx_v[s, pl.ds(g * L, L)] = plsc.load_gather(
                idx_v, [rows[g], t_vec])
        pltpu.async_copy(tab_hbm.at[pidx_v.at[s]], pbuf.at[s], gsems.at[s])

    def wait_gather(s):
        pltpu.make_async_copy(
            tab_hbm.at[pidx_v.at[0]], pbuf.at[s], gsems.at[s]).wait()

    def start_write(t, s):
        pltpu.async_copy(obuf.at[s], out_hbm.at[t, :, wid], wsems.at[s])

    def wait_write(s):
        pltpu.make_async_copy(
            obuf.at[0], out_hbm.at[0, :, 0], wsems.at[s]).wait()

    def transpose_scale(s):
        @pl.loop(0, D, unroll=8)
        def d_step(d):
            d_vec = jnp.full((L,), 0, jnp.int32) + d
            for g in range(NG):
                v = plsc.load_gather(pbuf.at[s], [rows[g], d_vec])
                obuf[s, d // 8, d % 8, pl.ds(g * L, L)] = v * SCALE

    start_gather(0, 0)

    @pl.loop(0, T, step=2)
    def outer(j0):
        for sl in range(2):
            t = j0 + sl
            nxt = (sl + 1) % 2

            @pl.when(t + 1 < T)
            def _():
                @pl.when(t >= 1)
                def _():
                    wait_write(nxt)
                start_gather(t + 1, nxt)

            wait_gather(sl)
            transpose_scale(sl)
            start_write(t, sl)

    wait_write(0)
    wait_write(1)


@jax.jit
def _emb_call(x, table):
    return pl.kernel(
        _emb_body,
        out_type=jax.ShapeDtypeStruct((T, D // 8, NW, 8, SB), jnp.float32),
        mesh=plsc.VectorSubcoreMesh(core_axis_name="c", subcore_axis_name="s"),
        scratch_types=[
            pltpu.VMEM((SB, T), jnp.int32),
            pltpu.VMEM((2, SB), jnp.int32),
            pltpu.VMEM((2, SB, D), jnp.float32),
            pltpu.VMEM((2, D // 8, 8, SB), jnp.float32),
            pltpu.SemaphoreType.DMA((2,)),
            pltpu.SemaphoreType.DMA((2,)),
        ],
        compiler_params=pltpu.CompilerParams(
            use_tc_tiling_on_sc=False, needs_layout_passes=False),
    )(x, table)


def kernel(x, table):
    out5 = _emb_call(x, table)
    return out5.transpose(2, 4, 0, 1, 3).reshape(S, T, D)


# speedup vs baseline: 1.5469x; 1.5469x over previous
import functools

import jax
import jax.numpy as jnp
from jax import lax
from jax.experimental import pallas as pl
from jax.experimental.pallas import tpu as pltpu
from jax.experimental.pallas import tpu_sc as plsc

S, T = 4096, 200
D = 64
SCALE = 8.0
NC, NS, L = 2, 16, 16
NW = NC * NS
ROWS_PER_W = S // NW
B_PER_W = ROWS_PER_W * T
G0 = 128
G1 = T - G0
NVEC = 13


def _emb_body(xf_hbm, tab_hbm, out_hbm, idx_v, pidx_v, pbuf, obuf,
              gsems, wsems):
    wid = lax.axis_index("s") * NC + lax.axis_index("c")
    base = wid * ROWS_PER_W

    pltpu.sync_copy(xf_hbm.at[pl.ds(base * T, B_PER_W)],
                    idx_v.at[pl.ds(0, B_PER_W)])

    def start_gather(r, sl):
        for c in range(NVEC):
            pidx_v[sl, pl.ds(c * L, L)] = (
                idx_v[pl.ds(r * T + c * L, L)] << 1)
        pltpu.async_copy(
            tab_hbm.at[pidx_v.at[sl, pl.ds(0, G0)]],
            pbuf.at[sl, pl.ds(0, G0)], gsems.at[sl])
        pltpu.async_copy(
            tab_hbm.at[pidx_v.at[sl, pl.ds(G0, G1)]],
            pbuf.at[sl, pl.ds(G0, G1)], gsems.at[sl])

    def wait_gather(sl):
        pltpu.make_async_copy(
            tab_hbm.at[pidx_v.at[0, pl.ds(0, G0)]],
            pbuf.at[0, pl.ds(0, G0)], gsems.at[sl]).wait()
        pltpu.make_async_copy(
            tab_hbm.at[pidx_v.at[0, pl.ds(G0, G1)]],
            pbuf.at[0, pl.ds(G0, G1)], gsems.at[sl]).wait()

    def start_write(r, sl):
        pltpu.async_copy(obuf.at[sl], out_hbm.at[base + r], wsems.at[sl])

    def wait_write(sl):
        pltpu.make_async_copy(obuf.at[0], out_hbm.at[0], wsems.at[sl]).wait()

    def scale_pack(sl):
        @pl.loop(0, T, unroll=8)
        def row_step(t):
            for c in range(D // L):
                csl = pl.ds(c * L, L)
                obuf[sl, t // 8, t % 8, csl] = pbuf[sl, t, csl] * SCALE

    start_gather(0, 0)

    @pl.loop(0, ROWS_PER_W, step=2)
    def outer(j0):
        for sl in range(2):
            r = j0 + sl
            nxt = (sl + 1) % 2

            @pl.when(r + 1 < ROWS_PER_W)
            def _():
                @pl.when(r >= 1)
                def _():
                    wait_write(nxt)
                start_gather(r + 1, nxt)

            wait_gather(sl)
            scale_pack(sl)
            start_write(r, sl)

    wait_write(0)
    wait_write(1)


@jax.jit
def _emb_call(xf, tab2):
    return pl.kernel(
        _emb_body,
        out_type=jax.ShapeDtypeStruct((S, T // 8, 8, 2 * D), jnp.float32),
        mesh=plsc.VectorSubcoreMesh(core_axis_name="c", subcore_axis_name="s"),
        scratch_types=[
            pltpu.VMEM((B_PER_W + L,), jnp.int32),
            pltpu.VMEM((2, NVEC * L), jnp.int32),
            pltpu.VMEM((2, T, D), jnp.float32),
            pltpu.VMEM((2, T // 8, 8, 2 * D), jnp.float32),
            pltpu.SemaphoreType.DMA((2,)),
            pltpu.SemaphoreType.DMA((2,)),
        ],
        compiler_params=pltpu.CompilerParams(use_tc_tiling_on_sc=False),
    )(xf, tab2)


def kernel(x, table):
    xf = x.reshape(S * T)
    tpad = jnp.pad(table, ((0, 0), (0, D)))
    tab2 = tpad.reshape(2 * 1000000, D)
    out4 = _emb_call(xf, tab2)
    return out4.reshape(S, T, 2 * D)[:, :, :D]


# speedup vs baseline: 2.3503x; 1.5194x over previous
import functools

import jax
import jax.numpy as jnp
from jax import lax
from jax.experimental import pallas as pl
from jax.experimental.pallas import tpu as pltpu
from jax.experimental.pallas import tpu_sc as plsc

S, T = 4096, 200
D = 64
SCALE = 8.0
NC, NS, L = 2, 16, 16
NW = NC * NS
ROWS_PER_W = S // NW
B_PER_W = ROWS_PER_W * T
G0 = 128
G1 = T - G0
NVEC = 13
N_BUF = 4
LOOK = 2


def _emb_body(xf_hbm, tab_hbm, out_hbm, idx_v, pidx_v, bufs, gsems, wsems):
    wid = lax.axis_index("s") * NC + lax.axis_index("c")
    base = wid * ROWS_PER_W

    pltpu.sync_copy(xf_hbm.at[pl.ds(base * T, B_PER_W)],
                    idx_v.at[pl.ds(0, B_PER_W)])

    def start_gather(r, sl):
        for c in range(NVEC):
            pidx_v[sl, pl.ds(c * L, L)] = (
                idx_v[pl.ds(r * T + c * L, L)] << 1)
        pltpu.async_copy(
            tab_hbm.at[pidx_v.at[sl, pl.ds(0, G0)]],
            bufs.at[sl, pl.ds(0, G0)], gsems.at[sl])
        pltpu.async_copy(
            tab_hbm.at[pidx_v.at[sl, pl.ds(G0, G1)]],
            bufs.at[sl, pl.ds(G0, G1)], gsems.at[sl])

    def wait_gather(sl):
        pltpu.make_async_copy(
            tab_hbm.at[pidx_v.at[0, pl.ds(0, G0)]],
            bufs.at[0, pl.ds(0, G0)], gsems.at[sl]).wait()
        pltpu.make_async_copy(
            tab_hbm.at[pidx_v.at[0, pl.ds(G0, G1)]],
            bufs.at[0, pl.ds(G0, G1)], gsems.at[sl]).wait()

    def start_write(r, sl):
        pltpu.async_copy(
            bufs.at[sl], out_hbm.at[base + r, :, pl.ds(0, D)], wsems.at[sl])

    def wait_write(sl):
        pltpu.make_async_copy(
            bufs.at[0], out_hbm.at[0, :, pl.ds(0, D)], wsems.at[sl]).wait()

    def scale_rows(sl):
        @pl.loop(0, T, unroll=8)
        def row_step(t):
            for c in range(D // L):
                csl = pl.ds(c * L, L)
                bufs[sl, t, csl] = bufs[sl, t, csl] * SCALE

    for r in range(LOOK):
        start_gather(r, r % N_BUF)

    @pl.loop(0, ROWS_PER_W, step=N_BUF)
    def outer(j0):
        for b in range(N_BUF):
            r = j0 + b
            bl = (b + LOOK) % N_BUF

            @pl.when(r + LOOK < ROWS_PER_W)
            def _():
                @pl.when(r + LOOK >= N_BUF)
                def _():
                    wait_write(bl)
                start_gather(r + LOOK, bl)

            wait_gather(b)
            scale_rows(b)
            start_write(r, b)

    for b in range(N_BUF):
        wait_write(b)


@jax.jit
def _emb_call(xf, tab2):
    return pl.kernel(
        _emb_body,
        out_type=jax.ShapeDtypeStruct((S, T, 2 * D), jnp.float32),
        mesh=plsc.VectorSubcoreMesh(core_axis_name="c", subcore_axis_name="s"),
        scratch_types=[
            pltpu.VMEM((B_PER_W + L,), jnp.int32),
            pltpu.VMEM((N_BUF, NVEC * L), jnp.int32),
            pltpu.VMEM((N_BUF, T, D), jnp.float32),
            pltpu.SemaphoreType.DMA((N_BUF,)),
            pltpu.SemaphoreType.DMA((N_BUF,)),
        ],
        compiler_params=pltpu.CompilerParams(use_tc_tiling_on_sc=False),
    )(xf, tab2)


def kernel(x, table):
    xf = x.reshape(S * T)
    tpad = jnp.pad(table, ((0, 0), (0, D)))
    tab2 = tpad.reshape(2 * 1000000, D)
    out3 = _emb_call(xf, tab2)
    return out3[:, :, :D]


# speedup vs baseline: 2.3510x; 1.0003x over previous
import functools

import jax
import jax.numpy as jnp
from jax import lax
from jax.experimental import pallas as pl
from jax.experimental.pallas import tpu as pltpu
from jax.experimental.pallas import tpu_sc as plsc

S, T = 4096, 200
D = 64
SCALE = 8.0
NC, NS, L = 2, 16, 16
NW = NC * NS
ROWS_PER_W = S // NW
B_PER_W = ROWS_PER_W * T
G0 = 128
G1 = T - G0
NVEC = 13
N_BUF = 8
LOOK = 3


def _emb_body(xf_hbm, tab_hbm, out_hbm, idx_v, pidx_v, bufs, gsems, wsems):
    wid = lax.axis_index("s") * NC + lax.axis_index("c")
    base = wid * ROWS_PER_W

    pltpu.sync_copy(xf_hbm.at[pl.ds(base * T, B_PER_W)],
                    idx_v.at[pl.ds(0, B_PER_W)])

    def start_gather(r, sl):
        for c in range(NVEC):
            pidx_v[sl, pl.ds(c * L, L)] = (
                idx_v[pl.ds(r * T + c * L, L)] << 1)
        pltpu.async_copy(
            tab_hbm.at[pidx_v.at[sl, pl.ds(0, G0)]],
            bufs.at[sl, pl.ds(0, G0)], gsems.at[sl])
        pltpu.async_copy(
            tab_hbm.at[pidx_v.at[sl, pl.ds(G0, G1)]],
            bufs.at[sl, pl.ds(G0, G1)], gsems.at[sl])

    def wait_gather(sl):
        pltpu.make_async_copy(
            tab_hbm.at[pidx_v.at[0, pl.ds(0, G0)]],
            bufs.at[0, pl.ds(0, G0)], gsems.at[sl]).wait()
        pltpu.make_async_copy(
            tab_hbm.at[pidx_v.at[0, pl.ds(G0, G1)]],
            bufs.at[0, pl.ds(G0, G1)], gsems.at[sl]).wait()

    def start_write(r, sl):
        pltpu.async_copy(
            bufs.at[sl], out_hbm.at[base + r, :, pl.ds(0, D)], wsems.at[sl])

    def wait_write(sl):
        pltpu.make_async_copy(
            bufs.at[0], out_hbm.at[0, :, pl.ds(0, D)], wsems.at[sl]).wait()

    def scale_rows(sl):
        @pl.loop(0, T, unroll=8)
        def row_step(t):
            for c in range(D // L):
                csl = pl.ds(c * L, L)
                bufs[sl, t, csl] = bufs[sl, t, csl] * SCALE

    for r in range(LOOK):
        start_gather(r, r % N_BUF)

    @pl.loop(0, ROWS_PER_W, step=N_BUF)
    def outer(j0):
        for b in range(N_BUF):
            r = j0 + b
            bl = (b + LOOK) % N_BUF

            @pl.when(r + LOOK < ROWS_PER_W)
            def _():
                @pl.when(r + LOOK >= N_BUF)
                def _():
                    wait_write(bl)
                start_gather(r + LOOK, bl)

            wait_gather(b)
            scale_rows(b)
            start_write(r, b)

    for b in range(N_BUF):
        wait_write(b)


@jax.jit
def _emb_call(xf, tab2):
    return pl.kernel(
        _emb_body,
        out_type=jax.ShapeDtypeStruct((S, T, 2 * D), jnp.float32),
        mesh=plsc.VectorSubcoreMesh(core_axis_name="c", subcore_axis_name="s"),
        scratch_types=[
            pltpu.VMEM((B_PER_W + L,), jnp.int32),
            pltpu.VMEM((N_BUF, NVEC * L), jnp.int32),
            pltpu.VMEM((N_BUF, T, D), jnp.float32),
            pltpu.SemaphoreType.DMA((N_BUF,)),
            pltpu.SemaphoreType.DMA((N_BUF,)),
        ],
        compiler_params=pltpu.CompilerParams(use_tc_tiling_on_sc=False),
    )(xf, tab2)


def kernel(x, table):
    xf = x.reshape(S * T)
    tpad = jnp.pad(table, ((0, 0), (0, D)))
    tab2 = tpad.reshape(2 * 1000000, D)
    out3 = _emb_call(xf, tab2)
    return out3[:, :, :D]
